# baseline (speedup 1.0000x reference)
import jax
import jax.numpy as jnp
from jax import lax
from jax.experimental import pallas as pl
from jax.experimental.pallas import tpu as pltpu
from jax.experimental.pallas import tpu_sc as plsc

N_NODES = 10000
N_EDGES = 10000
NNZ = 320000
D = 128

NC = 2
NS = 16
NW = NC * NS
CH = 128
CPW = 79
CPW1 = 103
CPW0 = 2 * CPW - CPW1
NNZP = NW * CPW * CH
NP = 10112
RPT = NP // NS
NPB = NP // 128
CNTW = NP + 128
CPR = CNTW // NS


def _sc_scatter_body(with_counts):
    def impl(table_hbm, gidx_hbm, sidx_hbm, zeros_hbm, acc_out, cnt_out,
             gb0, gb1, sb0, sb1, rows0, rows1, ones_v, zc_v, cnt_sh,
             gsem0, gsem1, ssem0, ssem1, isem0, isem1, csem0, csem1,
             acc_sh):
        c = lax.axis_index("c")
        s = lax.axis_index("s")
        w = c * NS + s
        cpw = jnp.where(c == 0, CPW0, CPW1)
        pairs = (cpw - 1) // 2
        pltpu.sync_copy(zeros_hbm.at[pl.ds(s * RPT, RPT)],
                        acc_sh.at[pl.ds(s * RPT, RPT)])
        if with_counts:
            for i in range(CH // 16):
                ones_v[pl.ds(i * 16, 16)] = jnp.full((16,), 1.0, jnp.float32)

            def fill(i, carry):
                zc_v[pl.ds(i * 16, 16)] = jnp.zeros((16,), jnp.float32)
                return carry
            lax.fori_loop(0, CPR // 16, fill, 0)
            pltpu.sync_copy(zc_v, cnt_sh.at[pl.ds(s * CPR, CPR)])
        plsc.subcore_barrier()

        def idx_load(j, gb, sb, isem):
            pltpu.async_copy(gidx_hbm.at[w, j, 0], gb, isem)
            pltpu.async_copy(sidx_hbm.at[w, j, 0], sb, isem)

        def idx_wait(gb, sb, isem):
            pltpu.make_async_copy(gidx_hbm.at[w, 0, 0], gb, isem).wait()
            pltpu.make_async_copy(sidx_hbm.at[w, 0, 0], sb, isem).wait()

        def gather(gb, rbuf, gsem):
            pltpu.async_copy(table_hbm.at[gb], rbuf, gsem)

        def gather_wait(gb, rbuf, gsem):
            pltpu.make_async_copy(table_hbm.at[gb], rbuf, gsem).wait()

        def scatter(sb, rbuf, ssem):
            pltpu.async_copy(rbuf, acc_sh.at[sb], ssem, add=True)

        def scatter_wait(sb, rbuf, ssem):
            pltpu.make_async_copy(rbuf, acc_sh.at[sb], ssem).wait()

        def count(sb, csem):
            if with_counts:
                pltpu.async_copy(ones_v, cnt_sh.at[sb], csem, add=True)

        def count_wait(sb, csem):
            if with_counts:
                pltpu.make_async_copy(ones_v, cnt_sh.at[sb], csem).wait()

        idx_load(0, gb0, sb0, isem0)
        idx_load(1, gb1, sb1, isem1)
        idx_wait(gb0, sb0, isem0)
        gather(gb0, rows0, gsem0)
        idx_wait(gb1, sb1, isem1)
        gather(gb1, rows1, gsem1)

        def pair(g, carry):
            j0 = 2 * g
            gather_wait(gb0, rows0, gsem0)
            scatter(sb0, rows0, ssem0)
            count(sb0, csem0)
            gather_wait(gb1, rows1, gsem1)
            scatter(sb1, rows1, ssem1)
            count(sb1, csem1)

            @pl.when(g < pairs - 1)
            def _():
                scatter_wait(sb0, rows0, ssem0)
                count_wait(sb0, csem0)
                idx_load(j0 + 2, gb0, sb0, isem0)
                idx_wait(gb0, sb0, isem0)
                gather(gb0, rows0, gsem0)
                scatter_wait(sb1, rows1, ssem1)
                count_wait(sb1, csem1)
                idx_load(j0 + 3, gb1, sb1, isem1)
                idx_wait(gb1, sb1, isem1)
                gather(gb1, rows1, gsem1)
            return carry

        lax.fori_loop(0, pairs, pair, 0)
        scatter_wait(sb0, rows0, ssem0)
        count_wait(sb0, csem0)
        scatter_wait(sb1, rows1, ssem1)
        count_wait(sb1, csem1)
        idx_load(cpw - 1, gb0, sb0, isem0)
        idx_wait(gb0, sb0, isem0)
        gather(gb0, rows0, gsem0)
        gather_wait(gb0, rows0, gsem0)
        pltpu.sync_copy(rows0, acc_sh.at[sb0], add=True)
        count(sb0, csem0)
        count_wait(sb0, csem0)
        plsc.subcore_barrier()
        pltpu.sync_copy(acc_sh.at[pl.ds(s * RPT, RPT)],
                        acc_out.at[c, pl.ds(s * RPT, RPT)])
        if with_counts:
            pltpu.sync_copy(cnt_sh.at[pl.ds(s * CPR, CPR)],
                            cnt_out.at[c, pl.ds(s * CPR, CPR)])

    if with_counts:
        def body(table_hbm, gidx_hbm, sidx_hbm, zeros_hbm, acc_out, cnt_out,
                 gb0, gb1, sb0, sb1, rows0, rows1, ones_v, zc_v, cnt_sh,
                 gsem0, gsem1, ssem0, ssem1, isem0, isem1, csem0, csem1,
                 acc_sh):
            impl(table_hbm, gidx_hbm, sidx_hbm, zeros_hbm, acc_out, cnt_out,
                 gb0, gb1, sb0, sb1, rows0, rows1, ones_v, zc_v, cnt_sh,
                 gsem0, gsem1, ssem0, ssem1, isem0, isem1, csem0, csem1,
                 acc_sh)
    else:
        def body(table_hbm, gidx_hbm, sidx_hbm, zeros_hbm, acc_out,
                 gb0, gb1, sb0, sb1, rows0, rows1,
                 gsem0, gsem1, ssem0, ssem1, isem0, isem1, acc_sh):
            impl(table_hbm, gidx_hbm, sidx_hbm, zeros_hbm, acc_out, None,
                 gb0, gb1, sb0, sb1, rows0, rows1, None, None, None,
                 gsem0, gsem1, ssem0, ssem1, isem0, isem1, None, None,
                 acc_sh)
    return body


def _sc_scatter_pass(table, gidx3d, sidx3d, zeros, with_counts):
    mesh = plsc.VectorSubcoreMesh(core_axis_name="c", subcore_axis_name="s",
                                  num_cores=NC, num_subcores=NS)
    outs = [jax.ShapeDtypeStruct((NC, NP, D), jnp.float32)]
    scratch = [
        pltpu.VMEM((CH,), jnp.int32),
        pltpu.VMEM((CH,), jnp.int32),
        pltpu.VMEM((CH,), jnp.int32),
        pltpu.VMEM((CH,), jnp.int32),
        pltpu.VMEM((CH, D), jnp.float32),
        pltpu.VMEM((CH, D), jnp.float32),
    ]
    if with_counts:
        outs.append(jax.ShapeDtypeStruct((NC, CNTW), jnp.float32))
        scratch += [pltpu.VMEM((CH,), jnp.float32),
                    pltpu.VMEM((CPR,), jnp.float32),
                    pltpu.VMEM_SHARED((CNTW,), jnp.float32)]
    scratch += [pltpu.SemaphoreType.DMA] * (8 if with_counts else 6)
    scratch += [
        pltpu.VMEM_SHARED((NP, D), jnp.float32),
    ]
    fn = pl.kernel(_sc_scatter_body(with_counts),
                   out_type=tuple(outs) if with_counts else outs[0],
                   mesh=mesh, scratch_types=scratch)
    return fn(table, gidx3d, sidx3d, zeros)


def _tc_matmul(xp, w):
    def body(x_ref, w_ref, o_ref):
        o_ref[...] = lax.dot_general(
            x_ref[...], w_ref[...], (((1,), (1,)), ((), ())),
            precision=lax.Precision.HIGHEST,
            preferred_element_type=jnp.float32)
    return pl.pallas_call(
        body,
        grid=(8,),
        in_specs=[pl.BlockSpec((NP // 8, D), lambda g: (g, 0)),
                  pl.BlockSpec((D, D), lambda g: (0, 0))],
        out_specs=pl.BlockSpec((NP // 8, D), lambda g: (g, 0)),
        out_shape=jax.ShapeDtypeStruct((NP, D), jnp.float32),
    )(xp, w)


def _tc_mean(esum_p, cnt_p):
    def body(e_ref, c_ref, o_ref):
        e = e_ref[0] + e_ref[1]
        cnt = c_ref[0] + c_ref[1]
        ri = lax.broadcasted_iota(jnp.int32, (128, 128), 0)
        ci = lax.broadcasted_iota(jnp.int32, (128, 128), 1)
        cb = jnp.broadcast_to(cnt[None, :], (128, 128))
        col = jnp.sum(jnp.where(ri == ci, cb, 0.0), axis=1, keepdims=True)
        o_ref[...] = e / jnp.maximum(col, 1.0)
    return pl.pallas_call(
        body,
        grid=(NPB,),
        in_specs=[pl.BlockSpec((NC, 128, D), lambda g: (0, g, 0)),
                  pl.BlockSpec((NC, 128), lambda g: (0, g))],
        out_specs=pl.BlockSpec((128, D), lambda g: (g, 0)),
        out_shape=jax.ShapeDtypeStruct((NP, D), jnp.float32),
    )(esum_p, cnt_p)


def _tc_combine(x2p, xv_p, eps):
    def body(x2_ref, xv_ref, eps_ref, o_ref):
        o_ref[...] = ((1.0 + eps_ref[0]) * x2_ref[...]
                      + xv_ref[0] + xv_ref[1])
    return pl.pallas_call(
        body,
        grid=(10,),
        in_specs=[pl.BlockSpec((1000, D), lambda g: (g, 0)),
                  pl.BlockSpec((NC, 1000, D), lambda g: (0, g, 0)),
                  pl.BlockSpec(memory_space=pltpu.SMEM)],
        out_specs=pl.BlockSpec((1000, D), lambda g: (g, 0)),
        out_shape=jax.ShapeDtypeStruct((N_NODES, D), jnp.float32),
    )(x2p, xv_p, eps)


def _split_chunks(idx, pad_id):
    pad = NNZP - NNZ
    chunks = jnp.concatenate(
        [idx, jnp.full((pad,), pad_id, jnp.int32)]).reshape(-1, 1, CH)
    n0 = NS * CPW0
    a = chunks[:n0].reshape(NS, CPW0, 1, CH)
    a = jnp.pad(a, ((0, 0), (0, CPW1 - CPW0), (0, 0), (0, 0)))
    b = chunks[n0:].reshape(NS, CPW1, 1, CH)
    return jnp.concatenate([a, b], axis=0)


def kernel(X, vertex, edges, W, eps):
    vp = _split_chunks(vertex, N_NODES)
    ep = _split_chunks(edges, N_EDGES)
    xp = jnp.pad(X, ((0, NP - N_NODES), (0, 0)))
    zeros = jnp.zeros((NP, D), jnp.float32)

    x2p = _tc_matmul(xp, W)
    esum_p, cnt_p = _sc_scatter_pass(x2p, vp, ep, zeros, True)
    xe_full = _tc_mean(esum_p, cnt_p)
    xv_p = _sc_scatter_pass(xe_full, ep, vp, zeros, False)
    xout = _tc_combine(x2p, xv_p, eps)
    return (xout, xe_full[:N_NODES])


# speedup vs baseline: 1.3052x; 1.3052x over previous
import jax
import jax.numpy as jnp
from jax import lax
from jax.experimental import pallas as pl
from jax.experimental.pallas import tpu as pltpu
from jax.experimental.pallas import tpu_sc as plsc

N_NODES = 10000
N_EDGES = 10000
NNZ = 320000
D = 128

NC = 2
NS = 16
NW = NC * NS
CH = 128
CPW = 79
NNZP = NW * CPW * CH
NP = 10112
RPT = NP // NS
NPB = NP // 128
CNTW = NP + 128
CPR = CNTW // NS
PAIRS = (CPW - 1) // 2


def _sc_scatter_body(with_counts):
    def impl(table_hbm, gidx_hbm, sidx_hbm, zeros_hbm, acc_out, cnt_out,
             gidx_v, sb0, sb1, rows0, rows1, ones_v, zc_v, cnt_sh,
             gsem0, gsem1, ssem0, ssem1, isem0, isem1, csem0, csem1,
             acc_sh):
        c = lax.axis_index("c")
        s = lax.axis_index("s")
        w = c * NS + s
        pltpu.sync_copy(gidx_hbm.at[w], gidx_v)
        pltpu.sync_copy(zeros_hbm.at[pl.ds(s * RPT, RPT)],
                        acc_sh.at[pl.ds(s * RPT, RPT)])
        if with_counts:
            for i in range(CH // 16):
                ones_v[pl.ds(i * 16, 16)] = jnp.full((16,), 1.0, jnp.float32)

            def fill(i, carry):
                zc_v[pl.ds(i * 16, 16)] = jnp.zeros((16,), jnp.float32)
                return carry
            lax.fori_loop(0, CPR // 16, fill, 0)
            pltpu.sync_copy(zc_v, cnt_sh.at[pl.ds(s * CPR, CPR)])
        plsc.subcore_barrier()

        def sidx_load(j, sb, isem):
            pltpu.async_copy(sidx_hbm.at[w, j, 0], sb, isem)

        def sidx_wait(sb, isem):
            pltpu.make_async_copy(sidx_hbm.at[w, 0, 0], sb, isem).wait()

        def gather(j, rbuf, gsem):
            pltpu.async_copy(table_hbm.at[gidx_v.at[j, 0]], rbuf, gsem)

        def gather_wait(rbuf, gsem):
            pltpu.make_async_copy(table_hbm.at[gidx_v.at[0, 0]], rbuf,
                                  gsem).wait()

        def scatter(sb, rbuf, ssem):
            pltpu.async_copy(rbuf, acc_sh.at[sb], ssem, add=True)

        def scatter_wait(sb, rbuf, ssem):
            pltpu.make_async_copy(rbuf, acc_sh.at[sb], ssem).wait()

        def count(sb, csem):
            if with_counts:
                pltpu.async_copy(ones_v, cnt_sh.at[sb], csem, add=True)

        def count_wait(sb, csem):
            if with_counts:
                pltpu.make_async_copy(ones_v, cnt_sh.at[sb], csem).wait()

        sidx_load(0, sb0, isem0)
        sidx_load(1, sb1, isem1)
        gather(0, rows0, gsem0)
        gather(1, rows1, gsem1)

        def pair(g, carry):
            j0 = 2 * g
            gather_wait(rows0, gsem0)
            sidx_wait(sb0, isem0)
            scatter(sb0, rows0, ssem0)
            count(sb0, csem0)
            gather_wait(rows1, gsem1)
            sidx_wait(sb1, isem1)
            scatter(sb1, rows1, ssem1)
            count(sb1, csem1)

            @pl.when(g < PAIRS - 1)
            def _():
                scatter_wait(sb0, rows0, ssem0)
                count_wait(sb0, csem0)
                sidx_load(j0 + 2, sb0, isem0)
                gather(j0 + 2, rows0, gsem0)
                scatter_wait(sb1, rows1, ssem1)
                count_wait(sb1, csem1)
                sidx_load(j0 + 3, sb1, isem1)
                gather(j0 + 3, rows1, gsem1)
            return carry

        lax.fori_loop(0, PAIRS, pair, 0)
        scatter_wait(sb0, rows0, ssem0)
        count_wait(sb0, csem0)
        scatter_wait(sb1, rows1, ssem1)
        count_wait(sb1, csem1)
        sidx_load(CPW - 1, sb0, isem0)
        gather(CPW - 1, rows0, gsem0)
        gather_wait(rows0, gsem0)
        sidx_wait(sb0, isem0)
        pltpu.sync_copy(rows0, acc_sh.at[sb0], add=True)
        count(sb0, csem0)
        count_wait(sb0, csem0)
        plsc.subcore_barrier()
        pltpu.sync_copy(acc_sh.at[pl.ds(s * RPT, RPT)],
                        acc_out.at[c, pl.ds(s * RPT, RPT)])
        if with_counts:
            pltpu.sync_copy(cnt_sh.at[pl.ds(s * CPR, CPR)],
                            cnt_out.at[c, pl.ds(s * CPR, CPR)])

    if with_counts:
        def body(table_hbm, gidx_hbm, sidx_hbm, zeros_hbm, acc_out, cnt_out,
                 gidx_v, sb0, sb1, rows0, rows1, ones_v, zc_v, cnt_sh,
                 gsem0, gsem1, ssem0, ssem1, isem0, isem1, csem0, csem1,
                 acc_sh):
            impl(table_hbm, gidx_hbm, sidx_hbm, zeros_hbm, acc_out, cnt_out,
                 gidx_v, sb0, sb1, rows0, rows1, ones_v, zc_v, cnt_sh,
                 gsem0, gsem1, ssem0, ssem1, isem0, isem1, csem0, csem1,
                 acc_sh)
    else:
        def body(table_hbm, gidx_hbm, sidx_hbm, zeros_hbm, acc_out,
                 gidx_v, sb0, sb1, rows0, rows1,
                 gsem0, gsem1, ssem0, ssem1, isem0, isem1, acc_sh):
            impl(table_hbm, gidx_hbm, sidx_hbm, zeros_hbm, acc_out, None,
                 gidx_v, sb0, sb1, rows0, rows1, None, None, None,
                 gsem0, gsem1, ssem0, ssem1, isem0, isem1, None, None,
                 acc_sh)
    return body


def _sc_scatter_pass(table, gidx4d, sidx4d, zeros, with_counts):
    mesh = plsc.VectorSubcoreMesh(core_axis_name="c", subcore_axis_name="s",
                                  num_cores=NC, num_subcores=NS)
    outs = [jax.ShapeDtypeStruct((NC, NP, D), jnp.float32)]
    scratch = [
        pltpu.VMEM((CPW, 1, CH), jnp.int32),
        pltpu.VMEM((CH,), jnp.int32),
        pltpu.VMEM((CH,), jnp.int32),
        pltpu.VMEM((CH, D), jnp.float32),
        pltpu.VMEM((CH, D), jnp.float32),
    ]
    if with_counts:
        outs.append(jax.ShapeDtypeStruct((NC, CNTW), jnp.float32))
        scratch += [pltpu.VMEM((CH,), jnp.float32),
                    pltpu.VMEM((CPR,), jnp.float32),
                    pltpu.VMEM_SHARED((CNTW,), jnp.float32)]
    scratch += [pltpu.SemaphoreType.DMA] * (8 if with_counts else 6)
    scratch += [
        pltpu.VMEM_SHARED((NP, D), jnp.float32),
    ]
    fn = pl.kernel(_sc_scatter_body(with_counts),
                   out_type=tuple(outs) if with_counts else outs[0],
                   mesh=mesh, scratch_types=scratch)
    return fn(table, gidx4d, sidx4d, zeros)


def _tc_matmul(xp, w):
    def body(x_ref, w_ref, o_ref):
        o_ref[...] = lax.dot_general(
            x_ref[...], w_ref[...], (((1,), (1,)), ((), ())),
            precision=lax.Precision.HIGHEST,
            preferred_element_type=jnp.float32)
    return pl.pallas_call(
        body,
        grid=(8,),
        in_specs=[pl.BlockSpec((NP // 8, D), lambda g: (g, 0)),
                  pl.BlockSpec((D, D), lambda g: (0, 0))],
        out_specs=pl.BlockSpec((NP // 8, D), lambda g: (g, 0)),
        out_shape=jax.ShapeDtypeStruct((NP, D), jnp.float32),
    )(xp, w)


def _tc_mean(esum_p, cnt_p):
    def body(e_ref, c_ref, o_ref):
        e = e_ref[0] + e_ref[1]
        cnt = c_ref[0] + c_ref[1]
        ri = lax.broadcasted_iota(jnp.int32, (128, 128), 0)
        ci = lax.broadcasted_iota(jnp.int32, (128, 128), 1)
        cb = jnp.broadcast_to(cnt[None, :], (128, 128))
        col = jnp.sum(jnp.where(ri == ci, cb, 0.0), axis=1, keepdims=True)
        o_ref[...] = e / jnp.maximum(col, 1.0)
    return pl.pallas_call(
        body,
        grid=(NPB,),
        in_specs=[pl.BlockSpec((NC, 128, D), lambda g: (0, g, 0)),
                  pl.BlockSpec((NC, 128), lambda g: (0, g))],
        out_specs=pl.BlockSpec((128, D), lambda g: (g, 0)),
        out_shape=jax.ShapeDtypeStruct((NP, D), jnp.float32),
    )(esum_p, cnt_p)


def _tc_combine(x2p, xv_p, eps):
    def body(x2_ref, xv_ref, eps_ref, o_ref):
        o_ref[...] = ((1.0 + eps_ref[0]) * x2_ref[...]
                      + xv_ref[0] + xv_ref[1])
    return pl.pallas_call(
        body,
        grid=(10,),
        in_specs=[pl.BlockSpec((1000, D), lambda g: (g, 0)),
                  pl.BlockSpec((NC, 1000, D), lambda g: (0, g, 0)),
                  pl.BlockSpec(memory_space=pltpu.SMEM)],
        out_specs=pl.BlockSpec((1000, D), lambda g: (g, 0)),
        out_shape=jax.ShapeDtypeStruct((N_NODES, D), jnp.float32),
    )(x2p, xv_p, eps)


def kernel(X, vertex, edges, W, eps):
    pad = NNZP - NNZ
    vp = jnp.concatenate(
        [vertex, jnp.full((pad,), N_NODES, jnp.int32)]).reshape(NW, CPW, 1, CH)
    ep = jnp.concatenate(
        [edges, jnp.full((pad,), N_EDGES, jnp.int32)]).reshape(NW, CPW, 1, CH)
    xp = jnp.pad(X, ((0, NP - N_NODES), (0, 0)))
    zeros = jnp.zeros((NP, D), jnp.float32)

    x2p = _tc_matmul(xp, W)
    esum_p, cnt_p = _sc_scatter_pass(x2p, vp, ep, zeros, True)
    xe_full = _tc_mean(esum_p, cnt_p)
    xv_p = _sc_scatter_pass(xe_full, ep, vp, zeros, False)
    xout = _tc_combine(x2p, xv_p, eps)
    return (xout, xe_full[:N_NODES])
